# baseline (speedup 1.0000x reference)
import dataclasses
import functools

import jax
import jax.numpy as jnp
from jax import lax
from jax.experimental import pallas as pl
from jax.experimental.pallas import tpu as pltpu
from jax.experimental.pallas import tpu_sc as plsc

ALPHA = 0.2
NC = 2
NS = 16
NW = NC * NS
CB = 128


def _sc_params():
    cp = pltpu.CompilerParams()
    if "needs_layout_passes" in pltpu.CompilerParams.__dataclass_fields__:
        cp = dataclasses.replace(cp, needs_layout_passes=False)
    return cp


def _mesh():
    return plsc.VectorSubcoreMesh(
        core_axis_name="c", subcore_axis_name="s", num_cores=NC,
        num_subcores=NS)


def _chunk_range(wid, C):
    q, r = C // NW, C % NW
    start = wid * q + jnp.minimum(wid, r)
    count = q + jnp.where(wid < r, 1, 0)
    return start, count


def _pad_rows(N):
    return ((N + 128 * NS - 1) // (128 * NS)) * (128 * NS)


def _matmul_body(h_ref, w_ref, out_ref):
    out_ref[...] = lax.dot_general(
        h_ref[...], w_ref[...],
        dimension_numbers=(((1,), (0,)), ((), ())),
        preferred_element_type=jnp.float32,
        precision=lax.Precision.HIGHEST,
    )


def _tc_matmul(h, W):
    n, d_in = h.shape
    d_out = W.shape[1]
    return pl.pallas_call(
        _matmul_body,
        out_shape=jax.ShapeDtypeStruct((n, d_out), jnp.float32),
    )(h, W)


def _sc_edge_ev(label, src, dst, *, N, E, D):
    C = E // CB
    q = C // NW
    IW = (q + 1) * CB
    QMAX = q + 2 if q % 2 == 0 else q + 1
    NP = _pad_rows(N)
    RW = NP // NS
    mesh = _mesh()

    @functools.partial(
        pl.kernel,
        out_type=[jax.ShapeDtypeStruct((E,), jnp.float32),
                  jax.ShapeDtypeStruct((NC, NP), jnp.float32)],
        mesh=mesh,
        scratch_types=[
            pltpu.VMEM_SHARED((NP,), jnp.float32),
            pltpu.VMEM((IW,), jnp.int32),
            pltpu.VMEM((IW,), jnp.int32),
            pltpu.VMEM((CB, D), jnp.float32),
            pltpu.VMEM((CB, D), jnp.float32),
            pltpu.VMEM((CB, D), jnp.float32),
            pltpu.VMEM((CB, D), jnp.float32),
            pltpu.VMEM((IW,), jnp.float32),
            pltpu.VMEM((CB,), jnp.int32),
            pltpu.VMEM((CB,), jnp.int32),
            pltpu.VMEM((16, 16), jnp.float32),
            pltpu.VMEM((RW,), jnp.float32),
            pltpu.SemaphoreType.DMA,
            pltpu.SemaphoreType.DMA,
            pltpu.SemaphoreType.DMA,
            pltpu.SemaphoreType.DMA,
            pltpu.SemaphoreType.DMA,
            pltpu.SemaphoreType.DMA,
        ],
        compiler_params=_sc_params(),
    )
    def run(label_hbm, src_hbm, dst_hbm, ev_out, rs_out,
            rs_acc, sidx_all, didx_all, ls0, ls1, ld0, ld1,
            evall, sded0, sded1, mbuf, z1d, gs0, gs1, gd0, gd1, ar0, ar1):
        cid = lax.axis_index("c")
        sid = lax.axis_index("s")
        wid = sid * NC + cid
        start, count = _chunk_range(wid, C)
        base0 = start * CB
        zf = jnp.zeros((16,), jnp.float32)
        lane = lax.iota(jnp.int32, 16)

        @pl.loop(0, RW // 16)
        def _(r):
            z1d[pl.ds(r * 16, 16)] = zf

        pltpu.sync_copy(z1d, rs_acc.at[pl.ds(sid * RW, RW)])

        pltpu.sync_copy(src_hbm.at[pl.ds(base0, q * CB)],
                        sidx_all.at[pl.ds(0, q * CB)])
        pltpu.sync_copy(dst_hbm.at[pl.ds(base0, q * CB)],
                        didx_all.at[pl.ds(0, q * CB)])

        @pl.when(count > q)
        def _():
            pltpu.sync_copy(src_hbm.at[pl.ds(base0 + q * CB, CB)],
                            sidx_all.at[pl.ds(q * CB, CB)])
            pltpu.sync_copy(dst_hbm.at[pl.ds(base0 + q * CB, CB)],
                            didx_all.at[pl.ds(q * CB, CB)])

        plsc.subcore_barrier()

        def start_gather(j, lsb, ldb, sems, semd):
            pltpu.async_copy(
                label_hbm.at[sidx_all.at[pl.ds(j * CB, CB)]], lsb, sems)
            pltpu.async_copy(
                label_hbm.at[didx_all.at[pl.ds(j * CB, CB)]], ldb, semd)

        def wait_gather(lsb, ldb, sems, semd):
            pltpu.make_async_copy(
                label_hbm.at[sidx_all.at[pl.ds(0, CB)]], lsb, sems).wait()
            pltpu.make_async_copy(
                label_hbm.at[didx_all.at[pl.ds(0, CB)]], ldb, semd).wait()

        def compute(j, lsb, ldb, sded, ar):
            off = j * CB
            nh = D // 32

            @pl.loop(0, CB // 16, unroll=4)
            def _(g):
                for jj in range(16):
                    rr = g * 16 + jj
                    acc0 = lsb[rr, pl.ds(0, 16)] * ldb[rr, pl.ds(0, 16)]
                    acc1 = (lsb[rr, pl.ds(16, 16)]
                            * ldb[rr, pl.ds(16, 16)])
                    for c in range(1, nh):
                        acc0 += (lsb[rr, pl.ds(2 * c * 16, 16)]
                                 * ldb[rr, pl.ds(2 * c * 16, 16)])
                        acc1 += (lsb[rr, pl.ds((2 * c + 1) * 16, 16)]
                                 * ldb[rr, pl.ds((2 * c + 1) * 16, 16)])
                    mbuf[jj, pl.ds(0, 16)] = acc0 + acc1
                e16 = plsc.load_gather(
                    mbuf, [lane, jnp.zeros((16,), jnp.int32)])
                for k in range(1, 16):
                    e16 += plsc.load_gather(
                        mbuf, [lane, jnp.full((16,), k, jnp.int32)])
                e16 = jnp.where(e16 > 0, e16, ALPHA * e16)
                evall[pl.ds(off + g * 16, 16)] = jnp.exp(e16)

            for k in range(CB // 16):
                sded[pl.ds(k * 16, 16)] = sidx_all[pl.ds(off + k * 16, 16)]

            pltpu.async_copy(evall.at[pl.ds(off, CB)], rs_acc.at[sded], ar,
                             add=True)

        def wait_rs_add(sded, ar):
            pltpu.make_async_copy(evall.at[pl.ds(0, CB)], rs_acc.at[sded],
                                  ar).wait()

        start_gather(0, ls0, ld0, gs0, gd0)
        start_gather(1, ls1, ld1, gs1, gd1)

        @pl.loop(0, QMAX, step=2)
        def _(jj):
            for b, lsb, ldb, sems, semd, sded, ar in (
                    (0, ls0, ld0, gs0, gd0, sded0, ar0),
                    (1, ls1, ld1, gs1, gd1, sded1, ar1)):
                j = jj + b

                @pl.when(j < count)
                def _(j=j, lsb=lsb, ldb=ldb, sems=sems, semd=semd,
                      sded=sded, ar=ar):
                    wait_gather(lsb, ldb, sems, semd)

                    @pl.when(j >= 2)
                    def _():
                        wait_rs_add(sded, ar)

                    compute(j, lsb, ldb, sded, ar)

                    @pl.when(j + 2 < count)
                    def _():
                        start_gather(j + 2, lsb, ldb, sems, semd)

        wait_rs_add(sded0, ar0)
        wait_rs_add(sded1, ar1)

        pltpu.sync_copy(evall.at[pl.ds(0, q * CB)],
                        ev_out.at[pl.ds(base0, q * CB)])

        @pl.when(count > q)
        def _():
            pltpu.sync_copy(evall.at[pl.ds(q * CB, CB)],
                            ev_out.at[pl.ds(base0 + q * CB, CB)])

        plsc.subcore_barrier()
        pltpu.sync_copy(rs_acc.at[pl.ds(sid * RW, RW)],
                        rs_out.at[cid].at[pl.ds(sid * RW, RW)])

    return run(label, src, dst)


def _sc_aggregate(Wh, ev, src, dst, *, N, E, D):
    C = E // CB
    q = C // NW
    QMAX = q + 2 if q % 2 == 0 else q + 1
    NP = _pad_rows(N)
    RW = NP // NS
    ZR = 16
    mesh = _mesh()

    @functools.partial(
        pl.kernel,
        out_type=jax.ShapeDtypeStruct((NC, NP, D), jnp.float32),
        mesh=mesh,
        scratch_types=[
            pltpu.VMEM_SHARED((NP, D), jnp.float32),
            pltpu.VMEM((CB, D), jnp.float32),
            pltpu.VMEM((CB, D), jnp.float32),
            pltpu.VMEM((CB,), jnp.int32),
            pltpu.VMEM((CB,), jnp.int32),
            pltpu.VMEM((CB,), jnp.int32),
            pltpu.VMEM((CB,), jnp.int32),
            pltpu.VMEM((CB,), jnp.float32),
            pltpu.VMEM((CB,), jnp.float32),
            pltpu.VMEM((CB,), jnp.int32),
            pltpu.VMEM((CB,), jnp.int32),
            pltpu.VMEM((ZR, D), jnp.float32),
            pltpu.SemaphoreType.DMA,
            pltpu.SemaphoreType.DMA,
            pltpu.SemaphoreType.DMA,
            pltpu.SemaphoreType.DMA,
            pltpu.SemaphoreType.DMA,
            pltpu.SemaphoreType.DMA,
        ],
        compiler_params=_sc_params(),
    )
    def run(wh_hbm, ev_hbm, src_hbm, dst_hbm, hp_out,
            hp_acc, wh0, wh1, di0, di1, sd0, sd1, ev0, ev1, sc0, sc1, zwide,
            gw0, gw1, ic0, ic1, aw0, aw1):
        cid = lax.axis_index("c")
        sid = lax.axis_index("s")
        wid = sid * NC + cid
        start, count = _chunk_range(wid, C)
        zf = jnp.zeros((16,), jnp.float32)

        @pl.loop(0, ZR)
        def _(r):
            @pl.loop(0, D // 16)
            def _(c):
                zwide[r, pl.ds(c * 16, 16)] = zf

        @pl.loop(0, RW // ZR)
        def _(j):
            pltpu.sync_copy(zwide, hp_acc.at[pl.ds(sid * RW + j * ZR, ZR)])

        plsc.subcore_barrier()

        def idx_start(j, dib, sdb, evb, sem):
            gbase = (start + j) * CB
            pltpu.async_copy(dst_hbm.at[pl.ds(gbase, CB)], dib, sem)
            pltpu.async_copy(src_hbm.at[pl.ds(gbase, CB)], sdb, sem)
            pltpu.async_copy(ev_hbm.at[pl.ds(gbase, CB)], evb, sem)

        def idx_wait(dib, sdb, evb, sem):
            pltpu.make_async_copy(dst_hbm.at[pl.ds(0, CB)], dib, sem).wait()
            pltpu.make_async_copy(src_hbm.at[pl.ds(0, CB)], sdb, sem).wait()
            pltpu.make_async_copy(ev_hbm.at[pl.ds(0, CB)], evb, sem).wait()

        def gather_start(dib, whb, sem):
            pltpu.async_copy(wh_hbm.at[dib], whb, sem)

        def gather_wait(dib, whb, sem):
            pltpu.make_async_copy(wh_hbm.at[dib], whb, sem).wait()

        def add_wait(whb, scb, aw):
            pltpu.make_async_copy(whb, hp_acc.at[scb], aw).wait()

        bufs = ((wh0, di0, sd0, ev0, sc0, gw0, ic0, aw0),
                (wh1, di1, sd1, ev1, sc1, gw1, ic1, aw1))

        idx_start(0, di0, sd0, ev0, ic0)
        idx_start(1, di1, sd1, ev1, ic1)
        idx_wait(di0, sd0, ev0, ic0)
        gather_start(di0, wh0, gw0)

        @pl.loop(0, QMAX, step=2)
        def _(jj):
            for b in range(2):
                whb, dib, sdb, evb, scb, gw, ic, aw = bufs[b]
                whn, din, sdn, evn, scn, gwn, icn, awn = bufs[1 - b]
                j = jj + b

                @pl.when(j < count)
                def _(j=j, whb=whb, dib=dib, sdb=sdb, evb=evb, scb=scb,
                      gw=gw, ic=ic, whn=whn, din=din, sdn=sdn, evn=evn,
                      scn=scn, gwn=gwn, icn=icn, awn=awn, aw=aw):
                    gather_wait(dib, whb, gw)

                    @pl.loop(0, CB // 16, unroll=4)
                    def _(g):
                        ev16 = evb[pl.ds(g * 16, 16)]
                        for jl in range(16):
                            r = g * 16 + jl
                            s = ev16[jl]
                            for c in range(D // 16):
                                whb[r, pl.ds(c * 16, 16)] = (
                                    whb[r, pl.ds(c * 16, 16)] * s)

                    for k in range(CB // 16):
                        scb[pl.ds(k * 16, 16)] = sdb[pl.ds(k * 16, 16)]

                    pltpu.async_copy(whb, hp_acc.at[scb], aw, add=True)

                    @pl.when(j + 2 < count)
                    def _():
                        idx_start(j + 2, dib, sdb, evb, ic)

                    @pl.when(j + 1 < count)
                    def _():
                        idx_wait(din, sdn, evn, icn)

                        @pl.when(j >= 1)
                        def _():
                            add_wait(whn, scn, awn)

                        gather_start(din, whn, gwn)

        add_wait(wh0, sc0, aw0)
        add_wait(wh1, sc1, aw1)

        plsc.subcore_barrier()
        pltpu.sync_copy(hp_acc.at[pl.ds(sid * RW, RW)],
                        hp_out.at[cid].at[pl.ds(sid * RW, RW)])

    return run(Wh, ev, src, dst)


def _combine_body(hp_ref, rs_ref, out_ref):
    num = hp_ref[0] + hp_ref[1]
    den = rs_ref[0] + rs_ref[1]
    out_ref[...] = num / jnp.maximum(den, 1e-9)


def _tc_combine(hp, rs3, *, N, D, BN):
    return pl.pallas_call(
        _combine_body,
        grid=(N // BN,),
        in_specs=[pl.BlockSpec((NC, BN, D), lambda i: (0, i, 0)),
                  pl.BlockSpec((NC, BN, 1), lambda i: (0, i, 0))],
        out_specs=pl.BlockSpec((BN, D), lambda i: (i, 0)),
        out_shape=jax.ShapeDtypeStruct((N, D), jnp.float32),
    )(hp, rs3)


def kernel(h, edge_index, label, W):
    N, D = label.shape
    E = edge_index.shape[1]
    src = edge_index[0]
    dst = edge_index[1]

    Wh = _tc_matmul(h, W)
    ev, rs = _sc_edge_ev(label, src, dst, N=N, E=E, D=D)
    hp = _sc_aggregate(Wh, ev, src, dst, N=N, E=E, D=D)
    rs3 = rs.reshape(rs.shape[0], rs.shape[1], 1)
    return _tc_combine(hp, rs3, N=N, D=D, BN=2000)


# speedup vs baseline: 1.5648x; 1.5648x over previous
import dataclasses
import functools

import jax
import jax.numpy as jnp
from jax import lax
from jax.experimental import pallas as pl
from jax.experimental.pallas import tpu as pltpu
from jax.experimental.pallas import tpu_sc as plsc

ALPHA = 0.2
NC = 2
NS = 16
NW = NC * NS
CB = 128


def _sc_params():
    cp = pltpu.CompilerParams()
    if "needs_layout_passes" in pltpu.CompilerParams.__dataclass_fields__:
        cp = dataclasses.replace(cp, needs_layout_passes=False)
    return cp


def _mesh():
    return plsc.VectorSubcoreMesh(
        core_axis_name="c", subcore_axis_name="s", num_cores=NC,
        num_subcores=NS)


def _chunk_range(wid, C):
    q, r = C // NW, C % NW
    start = wid * q + jnp.minimum(wid, r)
    count = q + jnp.where(wid < r, 1, 0)
    return start, count


def _pad_rows(N):
    return ((N + 128 * NS - 1) // (128 * NS)) * (128 * NS)


def _matmul_body(h_ref, w_ref, out_ref):
    out_ref[...] = lax.dot_general(
        h_ref[...], w_ref[...],
        dimension_numbers=(((1,), (0,)), ((), ())),
        preferred_element_type=jnp.float32,
        precision=lax.Precision.HIGHEST,
    )


def _tc_matmul(h, W):
    n, d_in = h.shape
    d_out = W.shape[1]
    return pl.pallas_call(
        _matmul_body,
        out_shape=jax.ShapeDtypeStruct((n, d_out), jnp.float32),
    )(h, W)


def _sc_edge_ev(label, src, dst, *, N, E, D):
    C = E // CB
    q = C // NW
    IW = (q + 1) * CB
    QMAX = q + 2 if q % 2 == 0 else q + 1
    NP = _pad_rows(N)
    RW = NP // NS
    mesh = _mesh()

    @functools.partial(
        pl.kernel,
        out_type=[jax.ShapeDtypeStruct((E,), jnp.float32),
                  jax.ShapeDtypeStruct((NC, NP), jnp.float32)],
        mesh=mesh,
        scratch_types=[
            pltpu.VMEM_SHARED((NP,), jnp.float32),
            pltpu.VMEM((IW,), jnp.int32),
            pltpu.VMEM((IW,), jnp.int32),
            pltpu.VMEM((CB, D), jnp.float32),
            pltpu.VMEM((CB, D), jnp.float32),
            pltpu.VMEM((CB, D), jnp.float32),
            pltpu.VMEM((CB, D), jnp.float32),
            pltpu.VMEM((IW,), jnp.float32),
            pltpu.VMEM((CB,), jnp.int32),
            pltpu.VMEM((CB,), jnp.int32),
            pltpu.VMEM((16, 16), jnp.float32),
            pltpu.VMEM((RW,), jnp.float32),
            pltpu.SemaphoreType.DMA,
            pltpu.SemaphoreType.DMA,
            pltpu.SemaphoreType.DMA,
            pltpu.SemaphoreType.DMA,
            pltpu.SemaphoreType.DMA,
            pltpu.SemaphoreType.DMA,
        ],
        compiler_params=_sc_params(),
    )
    def run(label_hbm, src_hbm, dst_hbm, ev_out, rs_out,
            rs_acc, sidx_all, didx_all, ls0, ls1, ld0, ld1,
            evall, sded0, sded1, mbuf, z1d, gs0, gs1, gd0, gd1, ar0, ar1):
        cid = lax.axis_index("c")
        sid = lax.axis_index("s")
        wid = sid * NC + cid
        start, count = _chunk_range(wid, C)
        base0 = start * CB
        zf = jnp.zeros((16,), jnp.float32)
        lane = lax.iota(jnp.int32, 16)

        @pl.loop(0, RW // 16)
        def _(r):
            z1d[pl.ds(r * 16, 16)] = zf

        pltpu.sync_copy(z1d, rs_acc.at[pl.ds(sid * RW, RW)])

        pltpu.sync_copy(src_hbm.at[pl.ds(base0, q * CB)],
                        sidx_all.at[pl.ds(0, q * CB)])
        pltpu.sync_copy(dst_hbm.at[pl.ds(base0, q * CB)],
                        didx_all.at[pl.ds(0, q * CB)])

        @pl.when(count > q)
        def _():
            pltpu.sync_copy(src_hbm.at[pl.ds(base0 + q * CB, CB)],
                            sidx_all.at[pl.ds(q * CB, CB)])
            pltpu.sync_copy(dst_hbm.at[pl.ds(base0 + q * CB, CB)],
                            didx_all.at[pl.ds(q * CB, CB)])

        plsc.subcore_barrier()

        def start_gather(j, lsb, ldb, sems, semd):
            pltpu.async_copy(
                label_hbm.at[sidx_all.at[pl.ds(j * CB, CB)]], lsb, sems)
            pltpu.async_copy(
                label_hbm.at[didx_all.at[pl.ds(j * CB, CB)]], ldb, semd)

        def wait_gather(lsb, ldb, sems, semd):
            pltpu.make_async_copy(
                label_hbm.at[sidx_all.at[pl.ds(0, CB)]], lsb, sems).wait()
            pltpu.make_async_copy(
                label_hbm.at[didx_all.at[pl.ds(0, CB)]], ldb, semd).wait()

        def compute(j, lsb, ldb, sded, ar):
            off = j * CB
            nh = D // 32

            @pl.loop(0, CB // 16)
            def _(g):
                for jj in range(16):
                    rr = g * 16 + jj
                    acc0 = lsb[rr, pl.ds(0, 16)] * ldb[rr, pl.ds(0, 16)]
                    acc1 = (lsb[rr, pl.ds(16, 16)]
                            * ldb[rr, pl.ds(16, 16)])
                    for c in range(1, nh):
                        acc0 += (lsb[rr, pl.ds(2 * c * 16, 16)]
                                 * ldb[rr, pl.ds(2 * c * 16, 16)])
                        acc1 += (lsb[rr, pl.ds((2 * c + 1) * 16, 16)]
                                 * ldb[rr, pl.ds((2 * c + 1) * 16, 16)])
                    mbuf[jj, pl.ds(0, 16)] = acc0 + acc1
                e16 = plsc.load_gather(
                    mbuf, [lane, jnp.zeros((16,), jnp.int32)])
                for k in range(1, 16):
                    e16 += plsc.load_gather(
                        mbuf, [lane, jnp.full((16,), k, jnp.int32)])
                e16 = jnp.where(e16 > 0, e16, ALPHA * e16)
                evall[pl.ds(off + g * 16, 16)] = jnp.exp(e16)

            for k in range(CB // 16):
                sded[pl.ds(k * 16, 16)] = sidx_all[pl.ds(off + k * 16, 16)]

            pltpu.async_copy(evall.at[pl.ds(off, CB)], rs_acc.at[sded], ar,
                             add=True)

        def wait_rs_add(sded, ar):
            pltpu.make_async_copy(evall.at[pl.ds(0, CB)], rs_acc.at[sded],
                                  ar).wait()

        start_gather(0, ls0, ld0, gs0, gd0)
        start_gather(1, ls1, ld1, gs1, gd1)

        @pl.loop(0, QMAX, step=2)
        def _(jj):
            for b, lsb, ldb, sems, semd, sded, ar in (
                    (0, ls0, ld0, gs0, gd0, sded0, ar0),
                    (1, ls1, ld1, gs1, gd1, sded1, ar1)):
                j = jj + b

                @pl.when(j < count)
                def _(j=j, lsb=lsb, ldb=ldb, sems=sems, semd=semd,
                      sded=sded, ar=ar):
                    wait_gather(lsb, ldb, sems, semd)

                    @pl.when(j >= 2)
                    def _():
                        wait_rs_add(sded, ar)

                    compute(j, lsb, ldb, sded, ar)

                    @pl.when(j + 2 < count)
                    def _():
                        start_gather(j + 2, lsb, ldb, sems, semd)

        wait_rs_add(sded0, ar0)
        wait_rs_add(sded1, ar1)

        pltpu.sync_copy(evall.at[pl.ds(0, q * CB)],
                        ev_out.at[pl.ds(base0, q * CB)])

        @pl.when(count > q)
        def _():
            pltpu.sync_copy(evall.at[pl.ds(q * CB, CB)],
                            ev_out.at[pl.ds(base0 + q * CB, CB)])

        plsc.subcore_barrier()
        pltpu.sync_copy(rs_acc.at[pl.ds(sid * RW, RW)],
                        rs_out.at[cid].at[pl.ds(sid * RW, RW)])

    return run(label, src, dst)


def _sc_aggregate(Wh, ev, src, dst, *, N, E, D):
    C = E // CB
    q = C // NW
    QMAX = q + 2 if q % 2 == 0 else q + 1
    NP = _pad_rows(N)
    RW = NP // NS
    ZR = 16
    mesh = _mesh()

    @functools.partial(
        pl.kernel,
        out_type=jax.ShapeDtypeStruct((NC, NP, D), jnp.float32),
        mesh=mesh,
        scratch_types=[
            pltpu.VMEM_SHARED((NP, D), jnp.float32),
            pltpu.VMEM((CB, D), jnp.float32),
            pltpu.VMEM((CB, D), jnp.float32),
            pltpu.VMEM((CB,), jnp.int32),
            pltpu.VMEM((CB,), jnp.int32),
            pltpu.VMEM((CB,), jnp.int32),
            pltpu.VMEM((CB,), jnp.int32),
            pltpu.VMEM((CB,), jnp.float32),
            pltpu.VMEM((CB,), jnp.float32),
            pltpu.VMEM((CB,), jnp.int32),
            pltpu.VMEM((CB,), jnp.int32),
            pltpu.VMEM((ZR, D), jnp.float32),
            pltpu.SemaphoreType.DMA,
            pltpu.SemaphoreType.DMA,
            pltpu.SemaphoreType.DMA,
            pltpu.SemaphoreType.DMA,
            pltpu.SemaphoreType.DMA,
            pltpu.SemaphoreType.DMA,
        ],
        compiler_params=_sc_params(),
    )
    def run(wh_hbm, ev_hbm, src_hbm, dst_hbm, hp_out,
            hp_acc, wh0, wh1, di0, di1, sd0, sd1, ev0, ev1, sc0, sc1, zwide,
            gw0, gw1, ic0, ic1, aw0, aw1):
        cid = lax.axis_index("c")
        sid = lax.axis_index("s")
        wid = sid * NC + cid
        start, count = _chunk_range(wid, C)
        zf = jnp.zeros((16,), jnp.float32)

        @pl.loop(0, ZR)
        def _(r):
            @pl.loop(0, D // 16)
            def _(c):
                zwide[r, pl.ds(c * 16, 16)] = zf

        @pl.loop(0, RW // ZR)
        def _(j):
            pltpu.sync_copy(zwide, hp_acc.at[pl.ds(sid * RW + j * ZR, ZR)])

        plsc.subcore_barrier()

        def idx_start(j, dib, sdb, evb, sem):
            gbase = (start + j) * CB
            pltpu.async_copy(dst_hbm.at[pl.ds(gbase, CB)], dib, sem)
            pltpu.async_copy(src_hbm.at[pl.ds(gbase, CB)], sdb, sem)
            pltpu.async_copy(ev_hbm.at[pl.ds(gbase, CB)], evb, sem)

        def idx_wait(dib, sdb, evb, sem):
            pltpu.make_async_copy(dst_hbm.at[pl.ds(0, CB)], dib, sem).wait()
            pltpu.make_async_copy(src_hbm.at[pl.ds(0, CB)], sdb, sem).wait()
            pltpu.make_async_copy(ev_hbm.at[pl.ds(0, CB)], evb, sem).wait()

        def gather_start(dib, whb, sem):
            pltpu.async_copy(wh_hbm.at[dib], whb, sem)

        def gather_wait(dib, whb, sem):
            pltpu.make_async_copy(wh_hbm.at[dib], whb, sem).wait()

        def add_wait(whb, scb, aw):
            pltpu.make_async_copy(whb, hp_acc.at[scb], aw).wait()

        bufs = ((wh0, di0, sd0, ev0, sc0, gw0, ic0, aw0),
                (wh1, di1, sd1, ev1, sc1, gw1, ic1, aw1))

        idx_start(0, di0, sd0, ev0, ic0)
        idx_start(1, di1, sd1, ev1, ic1)
        idx_wait(di0, sd0, ev0, ic0)
        gather_start(di0, wh0, gw0)

        @pl.loop(0, QMAX, step=2)
        def _(jj):
            for b in range(2):
                whb, dib, sdb, evb, scb, gw, ic, aw = bufs[b]
                whn, din, sdn, evn, scn, gwn, icn, awn = bufs[1 - b]
                j = jj + b

                @pl.when(j < count)
                def _(j=j, whb=whb, dib=dib, sdb=sdb, evb=evb, scb=scb,
                      gw=gw, ic=ic, whn=whn, din=din, sdn=sdn, evn=evn,
                      scn=scn, gwn=gwn, icn=icn, awn=awn, aw=aw):
                    gather_wait(dib, whb, gw)

                    @pl.loop(0, CB // 16)
                    def _(g):
                        ev16 = evb[pl.ds(g * 16, 16)]
                        for jl in range(16):
                            r = g * 16 + jl
                            s = ev16[jl]
                            for c in range(D // 16):
                                whb[r, pl.ds(c * 16, 16)] = (
                                    whb[r, pl.ds(c * 16, 16)] * s)

                    for k in range(CB // 16):
                        scb[pl.ds(k * 16, 16)] = sdb[pl.ds(k * 16, 16)]

                    pltpu.async_copy(whb, hp_acc.at[scb], aw, add=True)

                    @pl.when(j + 2 < count)
                    def _():
                        idx_start(j + 2, dib, sdb, evb, ic)

                    @pl.when(j + 1 < count)
                    def _():
                        idx_wait(din, sdn, evn, icn)

                        @pl.when(j >= 1)
                        def _():
                            add_wait(whn, scn, awn)

                        gather_start(din, whn, gwn)

        add_wait(wh0, sc0, aw0)
        add_wait(wh1, sc1, aw1)

        plsc.subcore_barrier()
        pltpu.sync_copy(hp_acc.at[pl.ds(sid * RW, RW)],
                        hp_out.at[cid].at[pl.ds(sid * RW, RW)])

    return run(Wh, ev, src, dst)


def _combine_body(hp_ref, rs_ref, out_ref):
    num = hp_ref[0] + hp_ref[1]
    den = rs_ref[0] + rs_ref[1]
    out_ref[...] = num / jnp.maximum(den, 1e-9)


def _tc_combine(hp, rs3, *, N, D, BN):
    return pl.pallas_call(
        _combine_body,
        grid=(N // BN,),
        in_specs=[pl.BlockSpec((NC, BN, D), lambda i: (0, i, 0)),
                  pl.BlockSpec((NC, BN, 1), lambda i: (0, i, 0))],
        out_specs=pl.BlockSpec((BN, D), lambda i: (i, 0)),
        out_shape=jax.ShapeDtypeStruct((N, D), jnp.float32),
    )(hp, rs3)


def kernel(h, edge_index, label, W):
    N, D = label.shape
    E = edge_index.shape[1]
    src = edge_index[0]
    dst = edge_index[1]

    Wh = _tc_matmul(h, W)
    ev, rs = _sc_edge_ev(label, src, dst, N=N, E=E, D=D)
    hp = _sc_aggregate(Wh, ev, src, dst, N=N, E=E, D=D)
    rs3 = rs.reshape(rs.shape[0], rs.shape[1], 1)
    return _tc_combine(hp, rs3, N=N, D=D, BN=2000)


# speedup vs baseline: 1.5732x; 1.0053x over previous
import dataclasses
import functools

import jax
import jax.numpy as jnp
from jax import lax
from jax.experimental import pallas as pl
from jax.experimental.pallas import tpu as pltpu
from jax.experimental.pallas import tpu_sc as plsc

ALPHA = 0.2
NC = 2
NS = 16
NW = NC * NS
CB = 128


def _sc_params():
    cp = pltpu.CompilerParams()
    if "needs_layout_passes" in pltpu.CompilerParams.__dataclass_fields__:
        cp = dataclasses.replace(cp, needs_layout_passes=False)
    return cp


def _mesh():
    return plsc.VectorSubcoreMesh(
        core_axis_name="c", subcore_axis_name="s", num_cores=NC,
        num_subcores=NS)


def _chunk_range(wid, C):
    q, r = C // NW, C % NW
    start = wid * q + jnp.minimum(wid, r)
    count = q + jnp.where(wid < r, 1, 0)
    return start, count


def _pad_rows(N):
    return ((N + 128 * NS - 1) // (128 * NS)) * (128 * NS)


def _matmul_body(h_ref, w_ref, out_ref):
    out_ref[...] = lax.dot_general(
        h_ref[...], w_ref[...],
        dimension_numbers=(((1,), (0,)), ((), ())),
        preferred_element_type=jnp.float32,
        precision=lax.Precision.HIGHEST,
    )


def _tc_matmul(h, W):
    n, d_in = h.shape
    d_out = W.shape[1]
    return pl.pallas_call(
        _matmul_body,
        out_shape=jax.ShapeDtypeStruct((n, d_out), jnp.float32),
    )(h, W)


def _sc_edge_ev(label, src, dst, *, N, E, D):
    C = E // CB
    q = C // NW
    IW = (q + 1) * CB
    QMAX = q + 2 if q % 2 == 0 else q + 1
    NP = _pad_rows(N)
    RW = NP // NS
    mesh = _mesh()

    @functools.partial(
        pl.kernel,
        out_type=[jax.ShapeDtypeStruct((E,), jnp.float32),
                  jax.ShapeDtypeStruct((NC, NP), jnp.float32)],
        mesh=mesh,
        scratch_types=[
            pltpu.VMEM_SHARED((NP,), jnp.float32),
            pltpu.VMEM((2 * IW,), jnp.int32),
            pltpu.VMEM((2 * CB, D), jnp.float32),
            pltpu.VMEM((2 * CB, D), jnp.float32),
            pltpu.VMEM((IW,), jnp.float32),
            pltpu.VMEM((CB,), jnp.int32),
            pltpu.VMEM((CB,), jnp.int32),
            pltpu.VMEM((16, 16), jnp.float32),
            pltpu.VMEM((RW,), jnp.float32),
            pltpu.SemaphoreType.DMA,
            pltpu.SemaphoreType.DMA,
            pltpu.SemaphoreType.DMA,
            pltpu.SemaphoreType.DMA,
            pltpu.SemaphoreType.DMA,
        ],
        compiler_params=_sc_params(),
    )
    def run(label_hbm, src_hbm, dst_hbm, ev_out, rs_out,
            rs_acc, cidx, cb0, cb1, evall, sded0, sded1, mbuf, z1d,
            fi, gs0, gs1, ar0, ar1):
        cid = lax.axis_index("c")
        sid = lax.axis_index("s")
        wid = sid * NC + cid
        start, count = _chunk_range(wid, C)
        base0 = start * CB
        zf = jnp.zeros((16,), jnp.float32)
        lane = lax.iota(jnp.int32, 16)

        @pl.loop(0, RW // 16)
        def _(r):
            z1d[pl.ds(r * 16, 16)] = zf

        pltpu.sync_copy(z1d, rs_acc.at[pl.ds(sid * RW, RW)])

        @pl.loop(0, count)
        def _(j):
            base = base0 + j * CB
            pltpu.async_copy(src_hbm.at[pl.ds(base, CB)],
                             cidx.at[pl.ds(j * 2 * CB, CB)], fi)
            pltpu.async_copy(dst_hbm.at[pl.ds(base, CB)],
                             cidx.at[pl.ds(j * 2 * CB + CB, CB)], fi)

        @pl.loop(0, 2 * count)
        def _(j):
            pltpu.make_async_copy(src_hbm.at[pl.ds(0, CB)],
                                  cidx.at[pl.ds(0, CB)], fi).wait()

        plsc.subcore_barrier()

        def start_gather(j, cbuf, sem):
            pltpu.async_copy(
                label_hbm.at[cidx.at[pl.ds(j * 2 * CB, 2 * CB)]], cbuf, sem)

        def wait_gather(cbuf, sem):
            pltpu.make_async_copy(
                label_hbm.at[cidx.at[pl.ds(0, 2 * CB)]], cbuf, sem).wait()

        def compute(j, cbuf, sded, ar):
            off = j * CB
            nh = D // 32

            @pl.loop(0, CB // 16)
            def _(g):
                for jj in range(16):
                    rr = g * 16 + jj
                    dr = CB + rr
                    acc0 = cbuf[rr, pl.ds(0, 16)] * cbuf[dr, pl.ds(0, 16)]
                    acc1 = (cbuf[rr, pl.ds(16, 16)]
                            * cbuf[dr, pl.ds(16, 16)])
                    for c in range(1, nh):
                        acc0 += (cbuf[rr, pl.ds(2 * c * 16, 16)]
                                 * cbuf[dr, pl.ds(2 * c * 16, 16)])
                        acc1 += (cbuf[rr, pl.ds((2 * c + 1) * 16, 16)]
                                 * cbuf[dr, pl.ds((2 * c + 1) * 16, 16)])
                    mbuf[jj, pl.ds(0, 16)] = acc0 + acc1
                e16 = plsc.load_gather(
                    mbuf, [lane, jnp.zeros((16,), jnp.int32)])
                for k in range(1, 16):
                    e16 += plsc.load_gather(
                        mbuf, [lane, jnp.full((16,), k, jnp.int32)])
                e16 = jnp.where(e16 > 0, e16, ALPHA * e16)
                evall[pl.ds(off + g * 16, 16)] = jnp.exp(e16)

            for k in range(CB // 16):
                sded[pl.ds(k * 16, 16)] = cidx[pl.ds(2 * off + k * 16, 16)]

            pltpu.async_copy(evall.at[pl.ds(off, CB)], rs_acc.at[sded], ar,
                             add=True)

        def wait_rs_add(sded, ar):
            pltpu.make_async_copy(evall.at[pl.ds(0, CB)], rs_acc.at[sded],
                                  ar).wait()

        start_gather(0, cb0, gs0)
        start_gather(1, cb1, gs1)

        @pl.loop(0, QMAX, step=2)
        def _(jj):
            for b, cbuf, sem, sded, ar in (
                    (0, cb0, gs0, sded0, ar0),
                    (1, cb1, gs1, sded1, ar1)):
                j = jj + b

                @pl.when(j < count)
                def _(j=j, cbuf=cbuf, sem=sem, sded=sded, ar=ar):
                    wait_gather(cbuf, sem)

                    @pl.when(j >= 2)
                    def _():
                        wait_rs_add(sded, ar)

                    compute(j, cbuf, sded, ar)

                    @pl.when(j + 2 < count)
                    def _():
                        start_gather(j + 2, cbuf, sem)

        wait_rs_add(sded0, ar0)
        wait_rs_add(sded1, ar1)

        pltpu.sync_copy(evall.at[pl.ds(0, q * CB)],
                        ev_out.at[pl.ds(base0, q * CB)])

        @pl.when(count > q)
        def _():
            pltpu.sync_copy(evall.at[pl.ds(q * CB, CB)],
                            ev_out.at[pl.ds(base0 + q * CB, CB)])

        plsc.subcore_barrier()
        pltpu.sync_copy(rs_acc.at[pl.ds(sid * RW, RW)],
                        rs_out.at[cid].at[pl.ds(sid * RW, RW)])

    return run(label, src, dst)


def _sc_aggregate(Wh, ev, src, dst, *, N, E, D):
    C = E // CB
    q = C // NW
    QMAX = q + 2 if q % 2 == 0 else q + 1
    NP = _pad_rows(N)
    RW = NP // NS
    ZR = 16
    mesh = _mesh()

    @functools.partial(
        pl.kernel,
        out_type=jax.ShapeDtypeStruct((NC, NP, D), jnp.float32),
        mesh=mesh,
        scratch_types=[
            pltpu.VMEM_SHARED((NP, D), jnp.float32),
            pltpu.VMEM((CB, D), jnp.float32),
            pltpu.VMEM((CB, D), jnp.float32),
            pltpu.VMEM((CB,), jnp.int32),
            pltpu.VMEM((CB,), jnp.int32),
            pltpu.VMEM((CB,), jnp.int32),
            pltpu.VMEM((CB,), jnp.int32),
            pltpu.VMEM((CB,), jnp.float32),
            pltpu.VMEM((CB,), jnp.float32),
            pltpu.VMEM((CB,), jnp.int32),
            pltpu.VMEM((CB,), jnp.int32),
            pltpu.VMEM((ZR, D), jnp.float32),
            pltpu.SemaphoreType.DMA,
            pltpu.SemaphoreType.DMA,
            pltpu.SemaphoreType.DMA,
            pltpu.SemaphoreType.DMA,
            pltpu.SemaphoreType.DMA,
            pltpu.SemaphoreType.DMA,
        ],
        compiler_params=_sc_params(),
    )
    def run(wh_hbm, ev_hbm, src_hbm, dst_hbm, hp_out,
            hp_acc, wh0, wh1, di0, di1, sd0, sd1, ev0, ev1, sc0, sc1, zwide,
            gw0, gw1, ic0, ic1, aw0, aw1):
        cid = lax.axis_index("c")
        sid = lax.axis_index("s")
        wid = sid * NC + cid
        start, count = _chunk_range(wid, C)
        zf = jnp.zeros((16,), jnp.float32)

        @pl.loop(0, ZR)
        def _(r):
            @pl.loop(0, D // 16)
            def _(c):
                zwide[r, pl.ds(c * 16, 16)] = zf

        @pl.loop(0, RW // ZR)
        def _(j):
            pltpu.sync_copy(zwide, hp_acc.at[pl.ds(sid * RW + j * ZR, ZR)])

        plsc.subcore_barrier()

        def idx_start(j, dib, sdb, evb, sem):
            gbase = (start + j) * CB
            pltpu.async_copy(dst_hbm.at[pl.ds(gbase, CB)], dib, sem)
            pltpu.async_copy(src_hbm.at[pl.ds(gbase, CB)], sdb, sem)
            pltpu.async_copy(ev_hbm.at[pl.ds(gbase, CB)], evb, sem)

        def idx_wait(dib, sdb, evb, sem):
            pltpu.make_async_copy(dst_hbm.at[pl.ds(0, CB)], dib, sem).wait()
            pltpu.make_async_copy(src_hbm.at[pl.ds(0, CB)], sdb, sem).wait()
            pltpu.make_async_copy(ev_hbm.at[pl.ds(0, CB)], evb, sem).wait()

        def gather_start(dib, whb, sem):
            pltpu.async_copy(wh_hbm.at[dib], whb, sem)

        def gather_wait(dib, whb, sem):
            pltpu.make_async_copy(wh_hbm.at[dib], whb, sem).wait()

        def add_wait(whb, scb, aw):
            pltpu.make_async_copy(whb, hp_acc.at[scb], aw).wait()

        bufs = ((wh0, di0, sd0, ev0, sc0, gw0, ic0, aw0),
                (wh1, di1, sd1, ev1, sc1, gw1, ic1, aw1))

        idx_start(0, di0, sd0, ev0, ic0)
        idx_start(1, di1, sd1, ev1, ic1)
        idx_wait(di0, sd0, ev0, ic0)
        gather_start(di0, wh0, gw0)

        @pl.loop(0, QMAX, step=2)
        def _(jj):
            for b in range(2):
                whb, dib, sdb, evb, scb, gw, ic, aw = bufs[b]
                whn, din, sdn, evn, scn, gwn, icn, awn = bufs[1 - b]
                j = jj + b

                @pl.when(j < count)
                def _(j=j, whb=whb, dib=dib, sdb=sdb, evb=evb, scb=scb,
                      gw=gw, ic=ic, whn=whn, din=din, sdn=sdn, evn=evn,
                      scn=scn, gwn=gwn, icn=icn, awn=awn, aw=aw):
                    gather_wait(dib, whb, gw)

                    @pl.loop(0, CB // 16)
                    def _(g):
                        ev16 = evb[pl.ds(g * 16, 16)]
                        for jl in range(16):
                            r = g * 16 + jl
                            s = ev16[jl]
                            for c in range(D // 16):
                                whb[r, pl.ds(c * 16, 16)] = (
                                    whb[r, pl.ds(c * 16, 16)] * s)

                    for k in range(CB // 16):
                        scb[pl.ds(k * 16, 16)] = sdb[pl.ds(k * 16, 16)]

                    pltpu.async_copy(whb, hp_acc.at[scb], aw, add=True)

                    @pl.when(j + 2 < count)
                    def _():
                        idx_start(j + 2, dib, sdb, evb, ic)

                    @pl.when(j + 1 < count)
                    def _():
                        idx_wait(din, sdn, evn, icn)

                        @pl.when(j >= 1)
                        def _():
                            add_wait(whn, scn, awn)

                        gather_start(din, whn, gwn)

        add_wait(wh0, sc0, aw0)
        add_wait(wh1, sc1, aw1)

        plsc.subcore_barrier()
        pltpu.sync_copy(hp_acc.at[pl.ds(sid * RW, RW)],
                        hp_out.at[cid].at[pl.ds(sid * RW, RW)])

    return run(Wh, ev, src, dst)


def _combine_body(hp_ref, rs_ref, out_ref):
    num = hp_ref[0] + hp_ref[1]
    den = rs_ref[0] + rs_ref[1]
    out_ref[...] = num / jnp.maximum(den, 1e-9)


def _tc_combine(hp, rs3, *, N, D, BN):
    return pl.pallas_call(
        _combine_body,
        grid=(N // BN,),
        in_specs=[pl.BlockSpec((NC, BN, D), lambda i: (0, i, 0)),
                  pl.BlockSpec((NC, BN, 1), lambda i: (0, i, 0))],
        out_specs=pl.BlockSpec((BN, D), lambda i: (i, 0)),
        out_shape=jax.ShapeDtypeStruct((N, D), jnp.float32),
    )(hp, rs3)


def kernel(h, edge_index, label, W):
    N, D = label.shape
    E = edge_index.shape[1]
    src = edge_index[0]
    dst = edge_index[1]

    Wh = _tc_matmul(h, W)
    ev, rs = _sc_edge_ev(label, src, dst, N=N, E=E, D=D)
    hp = _sc_aggregate(Wh, ev, src, dst, N=N, E=E, D=D)
    rs3 = rs.reshape(rs.shape[0], rs.shape[1], 1)
    return _tc_combine(hp, rs3, N=N, D=D, BN=2000)
